# initial kernel scaffold (unmeasured)
import jax
import jax.numpy as jnp
from jax import lax
from jax.experimental import pallas as pl
from jax.experimental.pallas import tpu as pltpu


def kernel(
    x,
):
    def body(*refs):
        pass

    out_shape = jax.ShapeDtypeStruct(..., jnp.float32)
    return pl.pallas_call(body, out_shape=out_shape)(...)


# baseline (device time: 53127 ns/iter reference)
import jax
import jax.numpy as jnp
from jax import lax
from jax.experimental import pallas as pl
from jax.experimental.pallas import tpu as pltpu


def kernel(x):
    m, n = x.shape

    def body(x_ref, out_ref, comm_ref, send_sem, recv_sem):
        my_x = lax.axis_index("x")
        my_y = lax.axis_index("y")
        peer = (1 - my_x, my_y)

        barrier_sem = pltpu.get_barrier_semaphore()
        pl.semaphore_signal(
            barrier_sem, inc=1,
            device_id=peer, device_id_type=pl.DeviceIdType.MESH,
        )
        pl.semaphore_wait(barrier_sem, 1)

        rdma = pltpu.make_async_remote_copy(
            src_ref=x_ref,
            dst_ref=comm_ref,
            send_sem=send_sem,
            recv_sem=recv_sem,
            device_id=peer,
            device_id_type=pl.DeviceIdType.MESH,
        )
        rdma.start()
        rdma.wait()

        out_ref[:, :] = x_ref[:, :] + comm_ref[:, :]

    return pl.pallas_call(
        body,
        out_shape=jax.ShapeDtypeStruct((m, n), x.dtype),
        in_specs=[pl.BlockSpec(memory_space=pltpu.VMEM)],
        out_specs=pl.BlockSpec(memory_space=pltpu.VMEM),
        scratch_shapes=[
            pltpu.VMEM((m, n), x.dtype),
            pltpu.SemaphoreType.DMA,
            pltpu.SemaphoreType.DMA,
        ],
        compiler_params=pltpu.CompilerParams(collective_id=0),
    )(x)


# device time: 35758 ns/iter; 1.4857x vs baseline; 1.4857x over previous
import jax
import jax.numpy as jnp
from jax import lax
from jax.experimental import pallas as pl
from jax.experimental.pallas import tpu as pltpu

C = 8


def kernel(x):
    m, n = x.shape
    half = m // 2
    ck = half // C

    def body(x_ref, out_ref, comm1, comm2, send1, recv1, send2, recv2):
        my_x = lax.axis_index("x")
        my_y = lax.axis_index("y")
        x_peer = (1 - my_x, my_y)
        y_peer = (my_x, 1 - my_y)

        barrier_sem = pltpu.get_barrier_semaphore()
        for nbr in (x_peer, y_peer):
            pl.semaphore_signal(
                barrier_sem, inc=1,
                device_id=nbr, device_id_type=pl.DeviceIdType.MESH,
            )
        pl.semaphore_wait(barrier_sem, 2)

        my_half = my_y * half
        other_half = (1 - my_y) * half

        rdma1 = []
        for i in range(C):
            r = pltpu.make_async_remote_copy(
                src_ref=x_ref.at[pl.ds(my_half + i * ck, ck)],
                dst_ref=comm1.at[i],
                send_sem=send1.at[i],
                recv_sem=recv1.at[i],
                device_id=x_peer,
                device_id_type=pl.DeviceIdType.MESH,
            )
            r.start()
            rdma1.append(r)

        rdma2 = []
        for i in range(C):
            rdma1[i].wait_recv()
            out_ref[pl.ds(my_half + i * ck, ck), :] = (
                x_ref[pl.ds(my_half + i * ck, ck), :] + comm1[i, :, :]
            )
            r = pltpu.make_async_remote_copy(
                src_ref=out_ref.at[pl.ds(my_half + i * ck, ck)],
                dst_ref=comm2.at[i],
                send_sem=send2.at[i],
                recv_sem=recv2.at[i],
                device_id=y_peer,
                device_id_type=pl.DeviceIdType.MESH,
            )
            r.start()
            rdma2.append(r)

        for i in range(C):
            rdma2[i].wait_recv()
            out_ref[pl.ds(other_half + i * ck, ck), :] = comm2[i, :, :]

        for i in range(C):
            rdma1[i].wait_send()
            rdma2[i].wait_send()

    return pl.pallas_call(
        body,
        out_shape=jax.ShapeDtypeStruct((m, n), x.dtype),
        in_specs=[pl.BlockSpec(memory_space=pltpu.VMEM)],
        out_specs=pl.BlockSpec(memory_space=pltpu.VMEM),
        scratch_shapes=[
            pltpu.VMEM((C, ck, n), x.dtype),
            pltpu.VMEM((C, ck, n), x.dtype),
            pltpu.SemaphoreType.DMA((C,)),
            pltpu.SemaphoreType.DMA((C,)),
            pltpu.SemaphoreType.DMA((C,)),
            pltpu.SemaphoreType.DMA((C,)),
        ],
        compiler_params=pltpu.CompilerParams(collective_id=0),
    )(x)


# device time: 34659 ns/iter; 1.5328x vs baseline; 1.0317x over previous
import jax
import jax.numpy as jnp
from jax import lax
from jax.experimental import pallas as pl
from jax.experimental.pallas import tpu as pltpu

C = 16


def kernel(x):
    m, n = x.shape
    half = m // 2
    ck = half // C

    def body(x_ref, out_ref, comm1, send1, recv1, send2, recv2):
        my_x = lax.axis_index("x")
        my_y = lax.axis_index("y")
        x_peer = (1 - my_x, my_y)
        y_peer = (my_x, 1 - my_y)

        barrier_sem = pltpu.get_barrier_semaphore()
        for nbr in (x_peer, y_peer):
            pl.semaphore_signal(
                barrier_sem, inc=1,
                device_id=nbr, device_id_type=pl.DeviceIdType.MESH,
            )
        pl.semaphore_wait(barrier_sem, 2)

        my_half = my_y * half

        rdma1 = []
        for i in range(C):
            r = pltpu.make_async_remote_copy(
                src_ref=x_ref.at[pl.ds(my_half + i * ck, ck)],
                dst_ref=comm1.at[i],
                send_sem=send1.at[i],
                recv_sem=recv1.at[i],
                device_id=x_peer,
                device_id_type=pl.DeviceIdType.MESH,
            )
            r.start()
            rdma1.append(r)

        rdma2 = []
        for i in range(C):
            rdma1[i].wait_recv()
            out_ref[pl.ds(my_half + i * ck, ck), :] = (
                x_ref[pl.ds(my_half + i * ck, ck), :] + comm1[i, :, :]
            )
            r = pltpu.make_async_remote_copy(
                src_ref=out_ref.at[pl.ds(my_half + i * ck, ck)],
                dst_ref=out_ref.at[pl.ds(my_half + i * ck, ck)],
                send_sem=send2.at[i],
                recv_sem=recv2.at[i],
                device_id=y_peer,
                device_id_type=pl.DeviceIdType.MESH,
            )
            r.start()
            rdma2.append(r)

        for i in range(C):
            rdma2[i].wait_recv()
        for i in range(C):
            rdma1[i].wait_send()
            rdma2[i].wait_send()

    return pl.pallas_call(
        body,
        out_shape=jax.ShapeDtypeStruct((m, n), x.dtype),
        in_specs=[pl.BlockSpec(memory_space=pltpu.VMEM)],
        out_specs=pl.BlockSpec(memory_space=pltpu.VMEM),
        scratch_shapes=[
            pltpu.VMEM((C, ck, n), x.dtype),
            pltpu.SemaphoreType.DMA((C,)),
            pltpu.SemaphoreType.DMA((C,)),
            pltpu.SemaphoreType.DMA((C,)),
            pltpu.SemaphoreType.DMA((C,)),
        ],
        compiler_params=pltpu.CompilerParams(collective_id=0),
    )(x)


# device time: 33667 ns/iter; 1.5780x vs baseline; 1.0295x over previous
import jax
import jax.numpy as jnp
from jax import lax
from jax.experimental import pallas as pl
from jax.experimental.pallas import tpu as pltpu

NC = 16
NE = 1
NF = NC - NE
NX = NC + NE


def kernel(x):
    m, n = x.shape
    half = m // 2
    ck = half // NC

    def body(x_ref, out_ref, comm1, comm2, send1, recv1, send2, recv2):
        my_x = lax.axis_index("x")
        my_y = lax.axis_index("y")
        x_peer = (1 - my_x, my_y)
        y_peer = (my_x, 1 - my_y)

        barrier_sem = pltpu.get_barrier_semaphore()
        for nbr in (x_peer, y_peer):
            pl.semaphore_signal(
                barrier_sem, inc=1,
                device_id=nbr, device_id_type=pl.DeviceIdType.MESH,
            )
        pl.semaphore_wait(barrier_sem, 2)

        my_half = my_y * half
        other_half = (1 - my_y) * half

        rdma1 = []
        for i in range(NX):
            if i < NC:
                src = x_ref.at[pl.ds(my_half + i * ck, ck)]
            else:
                j = NF + (i - NC)
                src = x_ref.at[pl.ds(other_half + j * ck, ck)]
            r = pltpu.make_async_remote_copy(
                src_ref=src,
                dst_ref=comm1.at[i],
                send_sem=send1.at[i],
                recv_sem=recv1.at[i],
                device_id=x_peer,
                device_id_type=pl.DeviceIdType.MESH,
            )
            r.start()
            rdma1.append(r)

        rdma2 = []
        for i in range(NC):
            rdma1[i].wait_recv()
            if i < NF:
                r = pltpu.make_async_remote_copy(
                    src_ref=comm1.at[i],
                    dst_ref=comm2.at[i],
                    send_sem=send2.at[i],
                    recv_sem=recv2.at[i],
                    device_id=y_peer,
                    device_id_type=pl.DeviceIdType.MESH,
                )
                r.start()
                rdma2.append(r)
            out_ref[pl.ds(my_half + i * ck, ck), :] = (
                x_ref[pl.ds(my_half + i * ck, ck), :] + comm1[i, :, :]
            )

        for i in range(NC, NX):
            j = NF + (i - NC)
            rdma1[i].wait_recv()
            out_ref[pl.ds(other_half + j * ck, ck), :] = (
                x_ref[pl.ds(other_half + j * ck, ck), :] + comm1[i, :, :]
            )

        for i in range(NF):
            rdma2[i].wait_recv()
            out_ref[pl.ds(other_half + i * ck, ck), :] = (
                x_ref[pl.ds(other_half + i * ck, ck), :] + comm2[i, :, :]
            )

        for r in rdma1:
            r.wait_send()
        for r in rdma2:
            r.wait_send()

    return pl.pallas_call(
        body,
        out_shape=jax.ShapeDtypeStruct((m, n), x.dtype),
        in_specs=[pl.BlockSpec(memory_space=pltpu.VMEM)],
        out_specs=pl.BlockSpec(memory_space=pltpu.VMEM),
        scratch_shapes=[
            pltpu.VMEM((NX, ck, n), x.dtype),
            pltpu.VMEM((NF, ck, n), x.dtype),
            pltpu.SemaphoreType.DMA((NX,)),
            pltpu.SemaphoreType.DMA((NX,)),
            pltpu.SemaphoreType.DMA((NF,)),
            pltpu.SemaphoreType.DMA((NF,)),
        ],
        compiler_params=pltpu.CompilerParams(collective_id=0),
    )(x)
